# baseline (speedup 1.0000x reference)
import functools

import jax
import jax.numpy as jnp
from jax import lax
from jax.experimental import pallas as pl
from jax.experimental.pallas import tpu as pltpu
from jax.experimental.pallas import tpu_sc as plsc

N = 10000
NP = 10240
E = 320000
CHUNK = 128
NC, NS = 2, 16
NW = NC * NS
ROWS = E // CHUNK
DG = 10
DNGROUPS = ROWS // DG
ZROWS = NP // NS
TROWS = N // NS

_MESH = plsc.VectorSubcoreMesh(core_axis_name="c", subcore_axis_name="s")
_SC_PARAMS = pltpu.CompilerParams(use_tc_tiling_on_sc=False,
                                  needs_layout_passes=False)


def _lane_bcast(v, l):
  idx = jnp.full((16,), l, jnp.int32)
  return lax.gather(
      v, idx[:, None],
      lax.GatherDimensionNumbers(offset_dims=(), collapsed_slice_dims=(0,),
                                 start_index_map=(0,)),
      slice_sizes=(1,), mode=lax.GatherScatterMode.PROMISE_IN_BOUNDS)


def _isqrt16(v):
  v = jnp.maximum(v, 1.0)
  bits = plsc.bitcast(v, jnp.int32)
  y = plsc.bitcast(jnp.int32(0x5F3759DF) - (bits >> 1), jnp.float32)
  for _ in range(3):
    y = y * (1.5 - 0.5 * v * y * y)
  return y


@functools.partial(
    pl.kernel,
    out_type=[
        jax.ShapeDtypeStruct((NC, NP), jnp.float32),
        jax.ShapeDtypeStruct((NC, NP // 8, 128), jnp.float32),
        jax.ShapeDtypeStruct((NP * 40 // 128, 128), jnp.float32),
    ],
    mesh=_MESH,
    compiler_params=_SC_PARAMS,
    scratch_types=[
        pltpu.VMEM((3, DG, CHUNK), jnp.int32),
        pltpu.VMEM((NP,), jnp.float32),
        pltpu.VMEM((ZROWS,), jnp.float32),
        pltpu.VMEM((ZROWS,), jnp.float32),
        pltpu.VMEM((ZROWS // 8, 128), jnp.float32),
        pltpu.VMEM((ZROWS * 40 // 128, 128), jnp.float32),
        pltpu.VMEM_SHARED((NS, NP), jnp.float32),
        pltpu.SemaphoreType.DMA,
    ],
)
def _deg_kernel(edges_hbm, zeros_hbm, isq_hbm, exp16_hbm,
                iexp40_hbm, idx_v, hist_v, dbuf, tbuf, ebuf, e40buf,
                part_sh, semI):
  c = lax.axis_index("c")
  s = lax.axis_index("s")
  ones16 = jnp.ones((16,), jnp.float32)

  pltpu.sync_copy(zeros_hbm, hist_v)

  def fire_idx(g, bb):
    pltpu.async_copy(edges_hbm.at[c, pl.ds(g * DG, DG)], idx_v.at[bb], semI)

  fire_idx(s, 0)

  @pl.loop(s, DNGROUPS, step=NS)
  def _group(g):
    t = (g - s) // NS
    b = lax.rem(t, 3)
    bn = lax.rem(t + 1, 3)
    pltpu.make_async_copy(edges_hbm.at[c, pl.ds(0, DG)], idx_v.at[b],
                          semI).wait()

    @pl.when(g + NS < DNGROUPS)
    def _():
      fire_idx(g + NS, bn)

    for j in range(DG):
      for k in range(CHUNK // 16):
        plsc.addupdate_scatter(
            hist_v, [idx_v[b, j, pl.ds(k * 16, 16)]], ones16)

  pltpu.sync_copy(hist_v, part_sh.at[s])
  plsc.subcore_barrier()

  pltpu.sync_copy(part_sh.at[0, pl.ds(s * ZROWS, ZROWS)], dbuf)
  for q in range(1, NS):
    pltpu.sync_copy(part_sh.at[q, pl.ds(s * ZROWS, ZROWS)], tbuf)
    for k in range(ZROWS // 16):
      dbuf[pl.ds(k * 16, 16)] = (dbuf[pl.ds(k * 16, 16)] +
                                 tbuf[pl.ds(k * 16, 16)])

  @pl.loop(0, ZROWS // 16)
  def _newton(k):
    dbuf[pl.ds(k * 16, 16)] = _isqrt16(dbuf[pl.ds(k * 16, 16)])

  pltpu.sync_copy(dbuf, isq_hbm.at[c, pl.ds(s * ZROWS, ZROWS)])

  @pl.loop(0, ZROWS // 16)
  def _exp16(k):
    v = dbuf[pl.ds(k * 16, 16)]
    for l in range(16):
      ebuf[2 * k + l // 8, pl.ds((l % 8) * 16, 16)] = _lane_bcast(v, l)

  pltpu.sync_copy(ebuf, exp16_hbm.at[c, pl.ds(s * (ZROWS // 8), ZROWS // 8)])

  @pl.when(c == 1)
  def _():
    lane_lt8 = lax.iota(jnp.int32, 16) < 8

    @pl.loop(0, ZROWS // 16)
    def _exp40(k):
      v = dbuf[pl.ds(k * 16, 16)]
      bl = [_lane_bcast(v, l) for l in range(16)]
      for m in range(40):
        j, kk = m // 5, m % 5
        if kk < 2:
          val = bl[2 * j]
        elif kk == 2:
          val = jnp.where(lane_lt8, bl[2 * j], bl[2 * j + 1])
        else:
          val = bl[2 * j + 1]
        e40buf[5 * k + m // 8, pl.ds((m % 8) * 16, 16)] = val

    pltpu.sync_copy(e40buf,
                    iexp40_hbm.at[pl.ds(s * (ZROWS * 40 // 128),
                                        ZROWS * 40 // 128)])


def _make_edge_pass16(F, G):
  NGROUPS = ROWS // G

  @functools.partial(
      pl.kernel,
      out_type=jax.ShapeDtypeStruct((NC, NP, F), jnp.float32),
      mesh=_MESH,
      compiler_params=_SC_PARAMS,
      scratch_types=[
          pltpu.VMEM((3, G, CHUNK), jnp.int32),
          pltpu.VMEM((3, G, CHUNK), jnp.int32),
          pltpu.VMEM((2, G, CHUNK, F), jnp.float32),
          pltpu.VMEM((ZROWS, F), jnp.float32),
          pltpu.VMEM((ZROWS,), jnp.float32),
          pltpu.VMEM_SHARED((N, F), jnp.float32),
          pltpu.VMEM_SHARED((NP, F), jnp.float32),
          pltpu.SemaphoreType.DMA,
          pltpu.SemaphoreType.DMA,
          pltpu.SemaphoreType.DMA,
      ],
  )
  def edge_pass(h_hbm, edges_hbm, isq_hbm, zeros_hbm, out_hbm,
                src_v, dst_v, rows_v, sbuf, scal_v, table_sh, agg_sh,
                semA, semS, semI):
    c = lax.axis_index("c")
    s = lax.axis_index("s")
    wid = s * NC + c

    pltpu.sync_copy(zeros_hbm.at[pl.ds(s * ZROWS, ZROWS)],
                    agg_sh.at[pl.ds(s * ZROWS, ZROWS)])

    def stage(nrows):
      base = s * ZROWS
      pltpu.sync_copy(h_hbm.at[pl.ds(base, nrows)], sbuf.at[pl.ds(0, nrows)])
      pltpu.sync_copy(isq_hbm.at[0, pl.ds(base, nrows)],
                      scal_v.at[pl.ds(0, nrows)])

      @pl.loop(0, nrows // 16)
      def _scale(k):
        v = scal_v[pl.ds(k * 16, 16)]
        for l in range(16):
          n = k * 16 + l
          sbuf[n] = sbuf[n] * _lane_bcast(v, l)

      pltpu.sync_copy(sbuf.at[pl.ds(0, nrows)],
                      table_sh.at[pl.ds(base, nrows)])

    @pl.when(s < NS - 1)
    def _():
      stage(ZROWS)

    @pl.when(s == NS - 1)
    def _():
      stage(N - (NS - 1) * ZROWS)

    plsc.subcore_barrier()

    def fire_idx(g, bb):
      base = g * G
      pltpu.async_copy(edges_hbm.at[0, pl.ds(base, G)], src_v.at[bb], semI)
      pltpu.async_copy(edges_hbm.at[1, pl.ds(base, G)], dst_v.at[bb], semI)

    fire_idx(wid, 0)

    @pl.loop(wid, NGROUPS, step=NW)
    def _group(g):
      t = (g - wid) // NW
      b = lax.rem(t, 3)
      bn = lax.rem(t + 1, 3)
      rp = lax.rem(t, 2)
      pltpu.make_async_copy(edges_hbm.at[0, pl.ds(0, G)], src_v.at[b],
                            semI).wait()
      pltpu.make_async_copy(edges_hbm.at[1, pl.ds(0, G)], dst_v.at[b],
                            semI).wait()

      @pl.when(t >= 2)
      def _():
        for j in range(G):
          pltpu.make_async_copy(rows_v.at[rp].at[j],
                                agg_sh.at[dst_v.at[bn].at[j]], semS).wait()

      for j in range(G):
        pltpu.async_copy(table_sh.at[src_v.at[b].at[j]], rows_v.at[rp].at[j],
                         semA)

      @pl.when(g + NW < NGROUPS)
      def _():
        fire_idx(g + NW, bn)

      for j in range(G):
        pltpu.make_async_copy(table_sh.at[src_v.at[b].at[j]],
                              rows_v.at[rp].at[j], semA).wait()
      for j in range(G):
        pltpu.async_copy(rows_v.at[rp].at[j], agg_sh.at[dst_v.at[b].at[j]],
                         semS, add=True)

    for bb in (0, 1):
      for j in range(G):
        pltpu.make_async_copy(rows_v.at[bb].at[j],
                              agg_sh.at[dst_v.at[bb].at[j]], semS).wait()

    plsc.subcore_barrier()
    pltpu.sync_copy(agg_sh.at[pl.ds(s * ZROWS, ZROWS)],
                    out_hbm.at[c, pl.ds(s * ZROWS, ZROWS)])

  return edge_pass


def _make_edge_pass40(F, G):
  NGROUPS = ROWS // G

  @functools.partial(
      pl.kernel,
      out_type=jax.ShapeDtypeStruct((NC, NP, F), jnp.float32),
      mesh=_MESH,
      compiler_params=_SC_PARAMS,
      scratch_types=[
          pltpu.VMEM((3, G, CHUNK), jnp.int32),
          pltpu.VMEM((3, G, CHUNK), jnp.int32),
          pltpu.VMEM((2, G, CHUNK, F), jnp.float32),
          pltpu.VMEM_SHARED((NP, F), jnp.float32),
          pltpu.VMEM_SHARED((NP, F), jnp.float32),
          pltpu.SemaphoreType.DMA,
          pltpu.SemaphoreType.DMA,
          pltpu.SemaphoreType.DMA,
      ],
  )
  def edge_pass(h_hbm, edges_hbm, zeros_hbm, out_hbm,
                src_v, dst_v, rows_v, table_sh, agg_sh, semA, semS, semI):
    c = lax.axis_index("c")
    s = lax.axis_index("s")
    wid = s * NC + c

    pltpu.sync_copy(zeros_hbm.at[pl.ds(s * ZROWS, ZROWS)],
                    agg_sh.at[pl.ds(s * ZROWS, ZROWS)])
    pltpu.sync_copy(h_hbm.at[pl.ds(s * ZROWS, ZROWS)],
                    table_sh.at[pl.ds(s * ZROWS, ZROWS)])
    plsc.subcore_barrier()

    def fire_idx(g, bb):
      base = g * G
      pltpu.async_copy(edges_hbm.at[0, pl.ds(base, G)], src_v.at[bb], semI)
      pltpu.async_copy(edges_hbm.at[1, pl.ds(base, G)], dst_v.at[bb], semI)

    fire_idx(wid, 0)

    @pl.loop(wid, NGROUPS, step=NW)
    def _group(g):
      t = (g - wid) // NW
      b = lax.rem(t, 3)
      bn = lax.rem(t + 1, 3)
      rp = lax.rem(t, 2)
      pltpu.make_async_copy(edges_hbm.at[0, pl.ds(0, G)], src_v.at[b],
                            semI).wait()
      pltpu.make_async_copy(edges_hbm.at[1, pl.ds(0, G)], dst_v.at[b],
                            semI).wait()

      @pl.when(t >= 2)
      def _():
        for j in range(G):
          pltpu.make_async_copy(rows_v.at[rp].at[j],
                                agg_sh.at[dst_v.at[bn].at[j]], semS).wait()

      for j in range(G):
        pltpu.async_copy(table_sh.at[src_v.at[b].at[j]], rows_v.at[rp].at[j],
                         semA)

      @pl.when(g + NW < NGROUPS)
      def _():
        fire_idx(g + NW, bn)

      for j in range(G):
        pltpu.make_async_copy(table_sh.at[src_v.at[b].at[j]],
                              rows_v.at[rp].at[j], semA).wait()
      for j in range(G):
        pltpu.async_copy(rows_v.at[rp].at[j], agg_sh.at[dst_v.at[b].at[j]],
                         semS, add=True)

    for bb in (0, 1):
      for j in range(G):
        pltpu.make_async_copy(rows_v.at[bb].at[j],
                              agg_sh.at[dst_v.at[bb].at[j]], semS).wait()

    plsc.subcore_barrier()
    pltpu.sync_copy(agg_sh.at[pl.ds(s * ZROWS, ZROWS)],
                    out_hbm.at[c, pl.ds(s * ZROWS, ZROWS)])

  return edge_pass


def _tc1_body(x_ref, w1_ref, o_ref):
  o_ref[...] = jnp.dot(x_ref[...], w1_ref[...],
                       preferred_element_type=jnp.float32)


def _tc2_body(p_ref, exp_ref, b1t_ref, w2big_ref, o_ref):
  h = (p_ref[0] + p_ref[1]) * exp_ref[1] + b1t_ref[...]
  h = jnp.maximum(h, 0.0) * exp_ref[0]
  o_ref[...] = jnp.dot(h, w2big_ref[...],
                       preferred_element_type=jnp.float32)


def _tc3_body(p_ref, iexp_ref, b2t_ref, o_ref):
  o_ref[...] = (p_ref[0] + p_ref[1]) * iexp_ref[...] + b2t_ref[...]


def _full(shape):
  return pl.BlockSpec(shape, lambda *_: tuple(0 for _ in shape))


def kernel(x, edge_index, W1, b1, W2, b2):
  f1 = W1.shape[1]
  f2 = W2.shape[1]
  r16 = NP * f1 // 128
  r40 = NP * f2 // 128

  edges = edge_index.astype(jnp.int32).reshape(2, ROWS, CHUNK)
  zerosN = jnp.zeros((NP,), jnp.float32)
  zeros1 = jnp.zeros((NP, f1), jnp.float32)
  zeros2 = jnp.zeros((NP, f2), jnp.float32)

  isq, exp16, iexp40 = _deg_kernel(edges, zerosN)

  h1 = pl.pallas_call(
      _tc1_body,
      grid=(2,),
      in_specs=[pl.BlockSpec((N // 2, x.shape[1]), lambda i: (i, 0)),
                _full(W1.shape)],
      out_specs=pl.BlockSpec((N // 2, f1), lambda i: (i, 0)),
      out_shape=jax.ShapeDtypeStruct((N, f1), jnp.float32),
  )(x, W1)

  agg1 = _make_edge_pass16(f1, 10)(h1, edges, isq, zeros1)

  w2big = jnp.kron(jnp.eye(8, dtype=jnp.float32), W2)
  b1t = jnp.tile(b1, 8)[None, :]
  h2 = pl.pallas_call(
      _tc2_body,
      grid=(1,),
      in_specs=[_full((NC, r16, 128)), _full((NC, r16, 128)),
                _full((1, 128)), _full((128, 8 * f2))],
      out_specs=_full((r16, 8 * f2)),
      out_shape=jax.ShapeDtypeStruct((r16, 8 * f2), jnp.float32),
  )(agg1.reshape(NC, r16, 128), exp16, b1t, w2big)

  h2t = h2.reshape(NP, f2)
  agg2 = _make_edge_pass40(f2, 5)(h2t, edges, zeros2)

  b2p5 = jnp.tile(b2, 16).reshape(5, 128)
  b2t = jnp.broadcast_to(b2p5[None], (r40 // 5, 5, 128)).reshape(r40, 128)
  out = pl.pallas_call(
      _tc3_body,
      grid=(1,),
      in_specs=[_full((NC, r40, 128)), _full((r40, 128)),
                _full((r40, 128))],
      out_specs=_full((r40, 128)),
      out_shape=jax.ShapeDtypeStruct((r40, 128), jnp.float32),
  )(agg2.reshape(NC, r40, 128), iexp40, b2t)

  return out.reshape(NP, f2)[:N]


# speedup vs baseline: 1.0359x; 1.0359x over previous
import functools

import jax
import jax.numpy as jnp
from jax import lax
from jax.experimental import pallas as pl
from jax.experimental.pallas import tpu as pltpu
from jax.experimental.pallas import tpu_sc as plsc

N = 10000
NP = 10240
E = 320000
CHUNK = 128
NC, NS = 2, 16
NW = NC * NS
ROWS = E // CHUNK
DG = 10
DNGROUPS = ROWS // DG
ZROWS = NP // NS
TROWS = N // NS

_MESH = plsc.VectorSubcoreMesh(core_axis_name="c", subcore_axis_name="s")
_SC_PARAMS = pltpu.CompilerParams(use_tc_tiling_on_sc=False,
                                  needs_layout_passes=False)


def _lane_bcast(v, l):
  idx = jnp.full((16,), l, jnp.int32)
  return lax.gather(
      v, idx[:, None],
      lax.GatherDimensionNumbers(offset_dims=(), collapsed_slice_dims=(0,),
                                 start_index_map=(0,)),
      slice_sizes=(1,), mode=lax.GatherScatterMode.PROMISE_IN_BOUNDS)


def _isqrt16(v):
  v = jnp.maximum(v, 1.0)
  bits = plsc.bitcast(v, jnp.int32)
  y = plsc.bitcast(jnp.int32(0x5F3759DF) - (bits >> 1), jnp.float32)
  for _ in range(3):
    y = y * (1.5 - 0.5 * v * y * y)
  return y


@functools.partial(
    pl.kernel,
    out_type=[
        jax.ShapeDtypeStruct((NC, NP), jnp.float32),
        jax.ShapeDtypeStruct((NC, NP // 8, 128), jnp.float32),
        jax.ShapeDtypeStruct((NP * 40 // 128, 128), jnp.float32),
    ],
    mesh=_MESH,
    compiler_params=_SC_PARAMS,
    scratch_types=[
        pltpu.VMEM((3, DG, CHUNK), jnp.int32),
        pltpu.VMEM((CHUNK,), jnp.float32),
        pltpu.VMEM((ZROWS,), jnp.float32),
        pltpu.VMEM((ZROWS // 8, 128), jnp.float32),
        pltpu.VMEM((ZROWS * 40 // 128, 128), jnp.float32),
        pltpu.VMEM_SHARED((NP,), jnp.float32),
        pltpu.SemaphoreType.DMA,
        pltpu.SemaphoreType.DMA,
    ],
)
def _deg_kernel(edges_hbm, ones_hbm, zeros_hbm, isq_hbm, exp16_hbm,
                iexp40_hbm, idx_v, ones_v, dbuf, ebuf, e40buf,
                deg_sh, semS, semI):
  c = lax.axis_index("c")
  s = lax.axis_index("s")

  pltpu.sync_copy(ones_hbm, ones_v)
  pltpu.sync_copy(zeros_hbm.at[pl.ds(s * ZROWS, ZROWS)],
                  deg_sh.at[pl.ds(s * ZROWS, ZROWS)])
  plsc.subcore_barrier()

  def fire_idx(g, bb):
    pltpu.async_copy(edges_hbm.at[c, pl.ds(g * DG, DG)], idx_v.at[bb], semI)

  fire_idx(s, 0)

  @pl.loop(s, DNGROUPS, step=NS)
  def _group(g):
    t = (g - s) // NS
    b = lax.rem(t, 3)
    bn = lax.rem(t + 1, 3)
    pltpu.make_async_copy(edges_hbm.at[c, pl.ds(0, DG)], idx_v.at[b],
                          semI).wait()

    @pl.when(t >= 2)
    def _():
      for j in range(DG):
        pltpu.make_async_copy(ones_v, deg_sh.at[idx_v.at[bn].at[j]],
                              semS).wait()

    @pl.when(g + NS < DNGROUPS)
    def _():
      fire_idx(g + NS, bn)

    for j in range(DG):
      pltpu.async_copy(ones_v, deg_sh.at[idx_v.at[b].at[j]], semS, add=True)

  for bb in (0, 1):
    for j in range(DG):
      pltpu.make_async_copy(ones_v, deg_sh.at[idx_v.at[bb].at[j]],
                            semS).wait()

  plsc.subcore_barrier()

  pltpu.sync_copy(deg_sh.at[pl.ds(s * ZROWS, ZROWS)], dbuf)

  @pl.loop(0, ZROWS // 16)
  def _newton(k):
    dbuf[pl.ds(k * 16, 16)] = _isqrt16(dbuf[pl.ds(k * 16, 16)])

  pltpu.sync_copy(dbuf, isq_hbm.at[c, pl.ds(s * ZROWS, ZROWS)])

  @pl.loop(0, ZROWS // 16)
  def _exp16(k):
    v = dbuf[pl.ds(k * 16, 16)]
    for l in range(16):
      ebuf[2 * k + l // 8, pl.ds((l % 8) * 16, 16)] = _lane_bcast(v, l)

  pltpu.sync_copy(ebuf, exp16_hbm.at[c, pl.ds(s * (ZROWS // 8), ZROWS // 8)])

  @pl.when(c == 1)
  def _():
    lane_lt8 = lax.iota(jnp.int32, 16) < 8

    @pl.loop(0, ZROWS // 16)
    def _exp40(k):
      v = dbuf[pl.ds(k * 16, 16)]
      bl = [_lane_bcast(v, l) for l in range(16)]
      for m in range(40):
        j, kk = m // 5, m % 5
        if kk < 2:
          val = bl[2 * j]
        elif kk == 2:
          val = jnp.where(lane_lt8, bl[2 * j], bl[2 * j + 1])
        else:
          val = bl[2 * j + 1]
        e40buf[5 * k + m // 8, pl.ds((m % 8) * 16, 16)] = val

    pltpu.sync_copy(e40buf,
                    iexp40_hbm.at[pl.ds(s * (ZROWS * 40 // 128),
                                        ZROWS * 40 // 128)])


def _make_edge_pass16(F, G):
  NGROUPS = ROWS // G

  @functools.partial(
      pl.kernel,
      out_type=jax.ShapeDtypeStruct((NC, NP, F), jnp.float32),
      mesh=_MESH,
      compiler_params=_SC_PARAMS,
      scratch_types=[
          pltpu.VMEM((3, G, CHUNK), jnp.int32),
          pltpu.VMEM((3, G, CHUNK), jnp.int32),
          pltpu.VMEM((2, G, CHUNK, F), jnp.float32),
          pltpu.VMEM((ZROWS, F), jnp.float32),
          pltpu.VMEM((ZROWS,), jnp.float32),
          pltpu.VMEM_SHARED((N, F), jnp.float32),
          pltpu.VMEM_SHARED((NP, F), jnp.float32),
          pltpu.SemaphoreType.DMA,
          pltpu.SemaphoreType.DMA,
          pltpu.SemaphoreType.DMA,
      ],
  )
  def edge_pass(h_hbm, edges_hbm, isq_hbm, zeros_hbm, out_hbm,
                src_v, dst_v, rows_v, sbuf, scal_v, table_sh, agg_sh,
                semA, semS, semI):
    c = lax.axis_index("c")
    s = lax.axis_index("s")
    wid = s * NC + c

    pltpu.sync_copy(zeros_hbm.at[pl.ds(s * ZROWS, ZROWS)],
                    agg_sh.at[pl.ds(s * ZROWS, ZROWS)])

    def stage(nrows):
      base = s * ZROWS
      pltpu.sync_copy(h_hbm.at[pl.ds(base, nrows)], sbuf.at[pl.ds(0, nrows)])
      pltpu.sync_copy(isq_hbm.at[0, pl.ds(base, nrows)],
                      scal_v.at[pl.ds(0, nrows)])

      @pl.loop(0, nrows // 16)
      def _scale(k):
        v = scal_v[pl.ds(k * 16, 16)]
        for l in range(16):
          n = k * 16 + l
          sbuf[n] = sbuf[n] * _lane_bcast(v, l)

      pltpu.sync_copy(sbuf.at[pl.ds(0, nrows)],
                      table_sh.at[pl.ds(base, nrows)])

    @pl.when(s < NS - 1)
    def _():
      stage(ZROWS)

    @pl.when(s == NS - 1)
    def _():
      stage(N - (NS - 1) * ZROWS)

    plsc.subcore_barrier()

    def fire_idx(g, bb):
      base = g * G
      pltpu.async_copy(edges_hbm.at[0, pl.ds(base, G)], src_v.at[bb], semI)
      pltpu.async_copy(edges_hbm.at[1, pl.ds(base, G)], dst_v.at[bb], semI)

    fire_idx(wid, 0)

    @pl.loop(wid, NGROUPS, step=NW)
    def _group(g):
      t = (g - wid) // NW
      b = lax.rem(t, 3)
      bn = lax.rem(t + 1, 3)
      rp = lax.rem(t, 2)
      pltpu.make_async_copy(edges_hbm.at[0, pl.ds(0, G)], src_v.at[b],
                            semI).wait()
      pltpu.make_async_copy(edges_hbm.at[1, pl.ds(0, G)], dst_v.at[b],
                            semI).wait()

      @pl.when(t >= 2)
      def _():
        for j in range(G):
          pltpu.make_async_copy(rows_v.at[rp].at[j],
                                agg_sh.at[dst_v.at[bn].at[j]], semS).wait()

      for j in range(G):
        pltpu.async_copy(table_sh.at[src_v.at[b].at[j]], rows_v.at[rp].at[j],
                         semA)

      @pl.when(g + NW < NGROUPS)
      def _():
        fire_idx(g + NW, bn)

      for j in range(G):
        pltpu.make_async_copy(table_sh.at[src_v.at[b].at[j]],
                              rows_v.at[rp].at[j], semA).wait()
      for j in range(G):
        pltpu.async_copy(rows_v.at[rp].at[j], agg_sh.at[dst_v.at[b].at[j]],
                         semS, add=True)

    for bb in (0, 1):
      for j in range(G):
        pltpu.make_async_copy(rows_v.at[bb].at[j],
                              agg_sh.at[dst_v.at[bb].at[j]], semS).wait()

    plsc.subcore_barrier()
    pltpu.sync_copy(agg_sh.at[pl.ds(s * ZROWS, ZROWS)],
                    out_hbm.at[c, pl.ds(s * ZROWS, ZROWS)])

  return edge_pass


def _make_edge_pass40(F, G):
  NGROUPS = ROWS // G

  @functools.partial(
      pl.kernel,
      out_type=jax.ShapeDtypeStruct((NC, NP, F), jnp.float32),
      mesh=_MESH,
      compiler_params=_SC_PARAMS,
      scratch_types=[
          pltpu.VMEM((3, G, CHUNK), jnp.int32),
          pltpu.VMEM((3, G, CHUNK), jnp.int32),
          pltpu.VMEM((2, G, CHUNK, F), jnp.float32),
          pltpu.VMEM_SHARED((NP, F), jnp.float32),
          pltpu.VMEM_SHARED((NP, F), jnp.float32),
          pltpu.SemaphoreType.DMA,
          pltpu.SemaphoreType.DMA,
          pltpu.SemaphoreType.DMA,
      ],
  )
  def edge_pass(h_hbm, edges_hbm, zeros_hbm, out_hbm,
                src_v, dst_v, rows_v, table_sh, agg_sh, semA, semS, semI):
    c = lax.axis_index("c")
    s = lax.axis_index("s")
    wid = s * NC + c

    pltpu.sync_copy(zeros_hbm.at[pl.ds(s * ZROWS, ZROWS)],
                    agg_sh.at[pl.ds(s * ZROWS, ZROWS)])
    pltpu.sync_copy(h_hbm.at[pl.ds(s * ZROWS, ZROWS)],
                    table_sh.at[pl.ds(s * ZROWS, ZROWS)])
    plsc.subcore_barrier()

    def fire_idx(g, bb):
      base = g * G
      pltpu.async_copy(edges_hbm.at[0, pl.ds(base, G)], src_v.at[bb], semI)
      pltpu.async_copy(edges_hbm.at[1, pl.ds(base, G)], dst_v.at[bb], semI)

    fire_idx(wid, 0)

    @pl.loop(wid, NGROUPS, step=NW)
    def _group(g):
      t = (g - wid) // NW
      b = lax.rem(t, 3)
      bn = lax.rem(t + 1, 3)
      rp = lax.rem(t, 2)
      pltpu.make_async_copy(edges_hbm.at[0, pl.ds(0, G)], src_v.at[b],
                            semI).wait()
      pltpu.make_async_copy(edges_hbm.at[1, pl.ds(0, G)], dst_v.at[b],
                            semI).wait()

      @pl.when(t >= 2)
      def _():
        for j in range(G):
          pltpu.make_async_copy(rows_v.at[rp].at[j],
                                agg_sh.at[dst_v.at[bn].at[j]], semS).wait()

      for j in range(G):
        pltpu.async_copy(table_sh.at[src_v.at[b].at[j]], rows_v.at[rp].at[j],
                         semA)

      @pl.when(g + NW < NGROUPS)
      def _():
        fire_idx(g + NW, bn)

      for j in range(G):
        pltpu.make_async_copy(table_sh.at[src_v.at[b].at[j]],
                              rows_v.at[rp].at[j], semA).wait()
      for j in range(G):
        pltpu.async_copy(rows_v.at[rp].at[j], agg_sh.at[dst_v.at[b].at[j]],
                         semS, add=True)

    for bb in (0, 1):
      for j in range(G):
        pltpu.make_async_copy(rows_v.at[bb].at[j],
                              agg_sh.at[dst_v.at[bb].at[j]], semS).wait()

    plsc.subcore_barrier()
    pltpu.sync_copy(agg_sh.at[pl.ds(s * ZROWS, ZROWS)],
                    out_hbm.at[c, pl.ds(s * ZROWS, ZROWS)])

  return edge_pass


def _tc1_body(x_ref, w1_ref, o_ref):
  o_ref[...] = jnp.dot(x_ref[...], w1_ref[...],
                       preferred_element_type=jnp.float32)


def _tc2_body(p_ref, exp_ref, b1t_ref, w2big_ref, o_ref):
  h = (p_ref[0] + p_ref[1]) * exp_ref[1] + b1t_ref[...]
  h = jnp.maximum(h, 0.0) * exp_ref[0]
  o_ref[...] = jnp.dot(h, w2big_ref[...],
                       preferred_element_type=jnp.float32)


def _tc3_body(p_ref, iexp_ref, b2t_ref, o_ref):
  o_ref[...] = (p_ref[0] + p_ref[1]) * iexp_ref[...] + b2t_ref[...]


def _full(shape):
  return pl.BlockSpec(shape, lambda *_: tuple(0 for _ in shape))


def kernel(x, edge_index, W1, b1, W2, b2):
  f1 = W1.shape[1]
  f2 = W2.shape[1]
  r16 = NP * f1 // 128
  r40 = NP * f2 // 128

  edges = edge_index.astype(jnp.int32).reshape(2, ROWS, CHUNK)
  ones128 = jnp.ones((CHUNK,), jnp.float32)
  zerosN = jnp.zeros((NP,), jnp.float32)
  zeros1 = jnp.zeros((NP, f1), jnp.float32)
  zeros2 = jnp.zeros((NP, f2), jnp.float32)

  isq, exp16, iexp40 = _deg_kernel(edges, ones128, zerosN)

  h1 = pl.pallas_call(
      _tc1_body,
      grid=(2,),
      in_specs=[pl.BlockSpec((N // 2, x.shape[1]), lambda i: (i, 0)),
                _full(W1.shape)],
      out_specs=pl.BlockSpec((N // 2, f1), lambda i: (i, 0)),
      out_shape=jax.ShapeDtypeStruct((N, f1), jnp.float32),
  )(x, W1)

  agg1 = _make_edge_pass16(f1, 10)(h1, edges, isq, zeros1)

  w2big = jnp.kron(jnp.eye(8, dtype=jnp.float32), W2)
  b1t = jnp.tile(b1, 8)[None, :]
  h2 = pl.pallas_call(
      _tc2_body,
      grid=(1,),
      in_specs=[_full((NC, r16, 128)), _full((NC, r16, 128)),
                _full((1, 128)), _full((128, 8 * f2))],
      out_specs=_full((r16, 8 * f2)),
      out_shape=jax.ShapeDtypeStruct((r16, 8 * f2), jnp.float32),
  )(agg1.reshape(NC, r16, 128), exp16, b1t, w2big)

  h2t = h2.reshape(NP, f2)
  agg2 = _make_edge_pass40(f2, 5)(h2t, edges, zeros2)

  b2p5 = jnp.tile(b2, 16).reshape(5, 128)
  b2t = jnp.broadcast_to(b2p5[None], (r40 // 5, 5, 128)).reshape(r40, 128)
  out = pl.pallas_call(
      _tc3_body,
      grid=(1,),
      in_specs=[_full((NC, r40, 128)), _full((r40, 128)),
                _full((r40, 128))],
      out_specs=_full((r40, 128)),
      out_shape=jax.ShapeDtypeStruct((r40, 128), jnp.float32),
  )(agg2.reshape(NC, r40, 128), iexp40, b2t)

  return out.reshape(NP, f2)[:N]
